# Initial kernel scaffold; baseline (speedup 1.0000x reference)
import jax
import jax.numpy as jnp
from jax import lax
from jax.experimental import pallas as pl
from jax.experimental.pallas import tpu as pltpu
from jax.experimental.pallas import tpu_sc as plsc

USER_NUM = 50000
ITEM_NUM = 50000
D = 64
H = 128
E = 800000
B = 4096

NC = 2
NS = 16
L = 16
NW = NC * NS

SUB = 128
EPT = 25600
E_PAD = EPT * NW
CHUNK_SUBS = 20
CHUNK = SUB * CHUNK_SUBS
NCHUNK = EPT // CHUNK

BLK = 2000
NBLK = USER_NUM // BLK

_HI = lax.Precision.HIGHEST


def _mlp(x, W1, b1, W2, b2):
    h = jnp.dot(x, W1, precision=_HI, preferred_element_type=jnp.float32) + b1
    h = jnp.where(h > 0, h, 0.2 * h)
    return jnp.dot(h, W2, precision=_HI, preferred_element_type=jnp.float32) + b2


def _loss_terms(logits, lab):
    l0 = logits[:, 0]
    l1 = logits[:, 1]
    m = jnp.maximum(l0, l1)
    logz = m + jnp.log(jnp.exp(l0 - m) + jnp.exp(l1 - m))
    ll = jnp.where(lab == 0, l0, l1)
    return jnp.sum(logz - ll)


def _items_body(x_ref, W1_ref, b1_ref, W2_ref, b2_ref, dW_ref,
                if1_ref, qT_ref):
    f = _mlp(x_ref[...], W1_ref[...], b1_ref[...], W2_ref[...], b2_ref[...])
    if1_ref[...] = f
    qT_ref[...] = lax.dot_general(
        dW_ref[...], f, (((0,), (1,)), ((), ())),
        precision=_HI, preferred_element_type=jnp.float32)


def _items_call(x, W1, b1, W2, b2, dW):
    return pl.pallas_call(
        _items_body,
        grid=(NBLK,),
        in_specs=[
            pl.BlockSpec((BLK, D), lambda i: (i, 0)),
            pl.BlockSpec((D, H), lambda i: (0, 0)),
            pl.BlockSpec((1, H), lambda i: (0, 0)),
            pl.BlockSpec((H, D), lambda i: (0, 0)),
            pl.BlockSpec((1, D), lambda i: (0, 0)),
            pl.BlockSpec((D, 2), lambda i: (0, 0)),
        ],
        out_specs=[
            pl.BlockSpec((BLK, D), lambda i: (i, 0)),
            pl.BlockSpec((2, BLK), lambda i: (0, i)),
        ],
        out_shape=[
            jax.ShapeDtypeStruct((ITEM_NUM, D), jnp.float32),
            jax.ShapeDtypeStruct((2, ITEM_NUM), jnp.float32),
        ],
    )(x, W1, b1, W2, b2, dW)


def _users_body(x_ref, W1_ref, b1_ref, W2_ref, b2_ref, dW_ref, db_ref,
                lab_ref, uf1_ref, lsum_ref):
    i = pl.program_id(0)
    f = _mlp(x_ref[...], W1_ref[...], b1_ref[...], W2_ref[...], b2_ref[...])
    uf1_ref[...] = f
    logits = jnp.dot(f, dW_ref[...], precision=_HI,
                     preferred_element_type=jnp.float32) + db_ref[...]
    part = _loss_terms(logits, lab_ref[:, 0])

    @pl.when(i == 0)
    def _():
        lsum_ref[0, 0] = 0.0

    lsum_ref[0, 0] += part


def _users_call(x, W1, b1, W2, b2, dW, db, labs):
    return pl.pallas_call(
        _users_body,
        grid=(NBLK,),
        in_specs=[
            pl.BlockSpec((BLK, D), lambda i: (i, 0)),
            pl.BlockSpec((D, H), lambda i: (0, 0)),
            pl.BlockSpec((1, H), lambda i: (0, 0)),
            pl.BlockSpec((H, D), lambda i: (0, 0)),
            pl.BlockSpec((1, D), lambda i: (0, 0)),
            pl.BlockSpec((D, 2), lambda i: (0, 0)),
            pl.BlockSpec((1, 2), lambda i: (0, 0)),
            pl.BlockSpec((BLK, 3), lambda i: (i, 0)),
        ],
        out_specs=[
            pl.BlockSpec((BLK, D), lambda i: (i, 0)),
            pl.BlockSpec((1, 1), lambda i: (0, 0)),
        ],
        out_shape=[
            jax.ShapeDtypeStruct((USER_NUM, D), jnp.float32),
            jax.ShapeDtypeStruct((1, 1), jnp.float32),
        ],
    )(x, W1, b1, W2, b2, dW, db, labs)


def _spmm_body(qT_hbm, col_hbm, row2_hbm, val_hbm, zeros_hbm, out_hbm,
               qT_v, col_v, row_v, val_v, y_v, acc_sh):
    c = lax.axis_index("c")
    s = lax.axis_index("s")
    wid = s * NC + c

    @pl.when(s == 0)
    def _():
        pltpu.sync_copy(zeros_hbm, acc_sh)

    pltpu.sync_copy(qT_hbm, qT_v)
    plsc.subcore_barrier()

    base_e = wid * EPT
    base_sub = wid * (EPT // SUB)
    zeros16 = jnp.zeros((L,), jnp.int32)
    ones16 = jnp.ones((L,), jnp.int32)
    iota16 = lax.iota(jnp.int32, L)
    gpsub = SUB // L

    def chunk_body(ci, _):
        e0 = base_e + ci * CHUNK
        pltpu.sync_copy(col_hbm.at[pl.ds(e0, CHUNK)], col_v)
        pltpu.sync_copy(val_hbm.at[pl.ds(e0, CHUNK)], val_v)
        pltpu.sync_copy(
            row2_hbm.at[pl.ds(base_sub + ci * CHUNK_SUBS, CHUNK_SUBS)], row_v)

        def group_body(g, _):
            col = col_v[pl.ds(g * L, L)]
            val = val_v[pl.ds(g * L, L)]
            g0 = plsc.load_gather(qT_v, [zeros16, col])
            g1 = plsc.load_gather(qT_v, [ones16, col])
            j = g // gpsub
            i0 = zeros16 + j
            i1 = (g % gpsub) * L + iota16
            plsc.store_scatter(y_v, [i0, i1, zeros16], g0 * val)
            plsc.store_scatter(y_v, [i0, i1, ones16], g1 * val)
            return 0

        lax.fori_loop(0, CHUNK // L, group_body, 0, unroll=4)

        def sub_body(j, _):
            pltpu.sync_copy(y_v.at[j], acc_sh.at[row_v.at[j]], add=True)
            return 0

        lax.fori_loop(0, CHUNK_SUBS, sub_body, 0)
        return 0

    lax.fori_loop(0, NCHUNK, chunk_body, 0)
    plsc.subcore_barrier()

    @pl.when(s == 0)
    def _():
        pltpu.sync_copy(acc_sh, out_hbm.at[c])


def _spmm_call(qT, col, row2, val, zeros):
    mesh = plsc.VectorSubcoreMesh(core_axis_name="c", subcore_axis_name="s")
    f = pl.kernel(
        _spmm_body,
        out_type=jax.ShapeDtypeStruct((NC, USER_NUM, 2), jnp.float32),
        mesh=mesh,
        scratch_types=[
            pltpu.VMEM((2, ITEM_NUM), jnp.float32),
            pltpu.VMEM((CHUNK,), jnp.int32),
            pltpu.VMEM((CHUNK_SUBS, SUB), jnp.int32),
            pltpu.VMEM((CHUNK,), jnp.float32),
            pltpu.VMEM((CHUNK_SUBS, SUB, 2), jnp.float32),
            pltpu.VMEM_SHARED((USER_NUM, 2), jnp.float32),
        ],
    )
    return f(qT, col, row2, val, zeros)


def _gather_body(uf1_hbm, if1_hbm, uidx_hbm, iidx_hbm, ub_hbm, ib_hbm,
                 idx_v, rows_v, sem):
    c = lax.axis_index("c")
    s = lax.axis_index("s")
    wid = s * NC + c
    bpt = B // NW
    base = wid * bpt

    pltpu.sync_copy(uidx_hbm.at[pl.ds(base, bpt)], idx_v)
    pltpu.async_copy(uf1_hbm.at[idx_v], rows_v, sem).wait()
    pltpu.sync_copy(rows_v, ub_hbm.at[pl.ds(base, bpt)])

    pltpu.sync_copy(iidx_hbm.at[pl.ds(base, bpt)], idx_v)
    pltpu.async_copy(if1_hbm.at[idx_v], rows_v, sem).wait()
    pltpu.sync_copy(rows_v, ib_hbm.at[pl.ds(base, bpt)])


def _gather_call(uf1, if1, uidx, iidx):
    mesh = plsc.VectorSubcoreMesh(core_axis_name="c", subcore_axis_name="s")
    f = pl.kernel(
        _gather_body,
        out_type=(
            jax.ShapeDtypeStruct((B, D), jnp.float32),
            jax.ShapeDtypeStruct((B, D), jnp.float32),
        ),
        mesh=mesh,
        scratch_types=[
            pltpu.VMEM((B // NW,), jnp.int32),
            pltpu.VMEM((B // NW, D), jnp.float32),
            pltpu.SemaphoreType.DMA,
        ],
    )
    return f(uf1, if1, uidx, iidx)


def _final_body(acc_ref, lab_ref, db_ref, dsum_ref, ub_ref, ib_ref, rat_ref,
                out_ref, lsum_ref):
    i = pl.program_id(0)

    @pl.when(i == 0)
    def _():
        lsum_ref[0, 0] = 0.0

    logits = acc_ref[0] + acc_ref[1] + db_ref[...]
    lsum_ref[0, 0] += _loss_terms(logits, lab_ref[:, 0])

    @pl.when(i == NBLK - 1)
    def _():
        d_loss1 = dsum_ref[0, 0] * (1.0 / USER_NUM)
        d_loss1_l = lsum_ref[0, 0] * (1.0 / USER_NUM)
        ub = ub_ref[...]
        ib = ib_ref[...]
        pred = jnp.sum(ub * ib, axis=1, keepdims=True)
        loss_part = jnp.mean((pred - rat_ref[...]) ** 2)
        l2 = 0.01 * (jnp.sum(ub * ub) + jnp.sum(ib * ib)) * (1.0 / B)
        loss_p_square = loss_part + l2
        d_loss_all = d_loss1 + 0.5 * d_loss1_l
        out_ref[0, 0] = d_loss_all
        out_ref[0, 1] = 10.0 * loss_p_square
        out_ref[0, 2] = -1000.0 * d_loss_all


def _final_call(acc, labs, db, dsum, ub, ib, rat):
    return pl.pallas_call(
        _final_body,
        grid=(NBLK,),
        in_specs=[
            pl.BlockSpec((NC, BLK, 2), lambda i: (0, i, 0)),
            pl.BlockSpec((BLK, 3), lambda i: (i, 0)),
            pl.BlockSpec((1, 2), lambda i: (0, 0)),
            pl.BlockSpec((1, 1), lambda i: (0, 0)),
            pl.BlockSpec((B, D), lambda i: (0, 0)),
            pl.BlockSpec((B, D), lambda i: (0, 0)),
            pl.BlockSpec((B, 1), lambda i: (0, 0)),
        ],
        out_specs=pl.BlockSpec((1, 3), lambda i: (0, 0)),
        out_shape=jax.ShapeDtypeStruct((1, 3), jnp.float32),
        scratch_shapes=[pltpu.SMEM((1, 1), jnp.float32)],
    )(acc, labs, db, dsum, ub, ib, rat)


def kernel(adj_indices, adj_values, user_batch, rating_batch, item_batch,
           users_features, gcn_user_embs, gcn_item_embs,
           f1_W1, f1_b1, f1_W2, f1_b2,
           f2_W1, f2_b1, f2_W2, f2_b2,
           f3_W1, f3_b1, f3_W2, f3_b2,
           d1_W, d1_b, d2_W, d2_b, d3_W, d3_b):
    row = adj_indices[0].astype(jnp.int32)
    col = adj_indices[1].astype(jnp.int32)
    pad = E_PAD - E
    row_p = jnp.concatenate([row, jnp.zeros((pad,), jnp.int32)])
    col_p = jnp.concatenate([col, jnp.zeros((pad,), jnp.int32)])
    val_p = jnp.concatenate([adj_values, jnp.zeros((pad,), jnp.float32)])
    row2 = row_p.reshape(E_PAD // SUB, SUB)

    b1 = f1_b1[None, :]
    b2 = f1_b2[None, :]
    db = d1_b[None, :]
    labs = users_features.astype(jnp.int32)
    zeros_acc = jnp.zeros((USER_NUM, 2), jnp.float32)

    if1, qT = _items_call(gcn_item_embs, f1_W1, b1, f1_W2, b2, d1_W)
    uf1, dsum = _users_call(gcn_user_embs, f1_W1, b1, f1_W2, b2, d1_W, db,
                            labs)
    acc = _spmm_call(qT, col_p, row2, val_p, zeros_acc)
    ub, ib = _gather_call(uf1, if1, user_batch.astype(jnp.int32),
                          item_batch.astype(jnp.int32))
    out = _final_call(acc, labs, db, dsum, ub, ib,
                      rating_batch[:, None])
    return out[0]


# speedup vs baseline: 22.3582x; 22.3582x over previous
import jax
import jax.numpy as jnp
from jax import lax
from jax.experimental import pallas as pl
from jax.experimental.pallas import tpu as pltpu
from jax.experimental.pallas import tpu_sc as plsc

USER_NUM = 50000
ITEM_NUM = 50000
D = 64
H = 128
E = 800000
B = 4096

NC = 2
NS = 16
L = 16
NW = NC * NS

SUB = 128
EPT = 25600
E_PAD = EPT * NW
CHUNK_SUBS = 8
CHUNK = SUB * CHUNK_SUBS
NCHUNK = EPT // CHUNK

BLK = 2000
NBLK = USER_NUM // BLK

_HI = lax.Precision.HIGHEST


def _mlp(x, W1, b1, W2, b2):
    h = jnp.dot(x, W1, precision=_HI, preferred_element_type=jnp.float32) + b1
    h = jnp.where(h > 0, h, 0.2 * h)
    return jnp.dot(h, W2, precision=_HI, preferred_element_type=jnp.float32) + b2


def _loss_terms(logits, lab):
    l0 = logits[:, 0]
    l1 = logits[:, 1]
    m = jnp.maximum(l0, l1)
    logz = m + jnp.log(jnp.exp(l0 - m) + jnp.exp(l1 - m))
    ll = jnp.where(lab == 0, l0, l1)
    return jnp.sum(logz - ll)


def _items_body(x_ref, W1_ref, b1_ref, W2_ref, b2_ref, dW_ref,
                if1_ref, q_ref):
    f = _mlp(x_ref[...], W1_ref[...], b1_ref[...], W2_ref[...], b2_ref[...])
    if1_ref[...] = jnp.concatenate([f, jnp.zeros_like(f)], axis=1)
    q_ref[...] = jnp.dot(f, dW_ref[...], precision=_HI,
                         preferred_element_type=jnp.float32)


def _items_call(x, W1, b1, W2, b2, dW):
    return pl.pallas_call(
        _items_body,
        grid=(NBLK,),
        in_specs=[
            pl.BlockSpec((BLK, D), lambda i: (i, 0)),
            pl.BlockSpec((D, H), lambda i: (0, 0)),
            pl.BlockSpec((1, H), lambda i: (0, 0)),
            pl.BlockSpec((H, D), lambda i: (0, 0)),
            pl.BlockSpec((1, D), lambda i: (0, 0)),
            pl.BlockSpec((D, 2), lambda i: (0, 0)),
        ],
        out_specs=[
            pl.BlockSpec((BLK, 2 * D), lambda i: (i, 0)),
            pl.BlockSpec((BLK, 2), lambda i: (i, 0)),
        ],
        out_shape=[
            jax.ShapeDtypeStruct((ITEM_NUM, 2 * D), jnp.float32),
            jax.ShapeDtypeStruct((ITEM_NUM, 2), jnp.float32),
        ],
    )(x, W1, b1, W2, b2, dW)


def _users_body(x_ref, W1_ref, b1_ref, W2_ref, b2_ref, dW_ref, db_ref,
                lab_ref, uf1_ref, lsum_ref):
    i = pl.program_id(0)
    f = _mlp(x_ref[...], W1_ref[...], b1_ref[...], W2_ref[...], b2_ref[...])
    uf1_ref[...] = jnp.concatenate([f, jnp.zeros_like(f)], axis=1)
    logits = jnp.dot(f, dW_ref[...], precision=_HI,
                     preferred_element_type=jnp.float32) + db_ref[...]
    part = _loss_terms(logits, lab_ref[:, 0])

    @pl.when(i == 0)
    def _():
        lsum_ref[...] = jnp.zeros((1, 1), jnp.float32)

    lsum_ref[...] += jnp.reshape(part, (1, 1))


def _users_call(x, W1, b1, W2, b2, dW, db, labs):
    return pl.pallas_call(
        _users_body,
        grid=(NBLK,),
        in_specs=[
            pl.BlockSpec((BLK, D), lambda i: (i, 0)),
            pl.BlockSpec((D, H), lambda i: (0, 0)),
            pl.BlockSpec((1, H), lambda i: (0, 0)),
            pl.BlockSpec((H, D), lambda i: (0, 0)),
            pl.BlockSpec((1, D), lambda i: (0, 0)),
            pl.BlockSpec((D, 2), lambda i: (0, 0)),
            pl.BlockSpec((1, 2), lambda i: (0, 0)),
            pl.BlockSpec((BLK, 3), lambda i: (i, 0)),
        ],
        out_specs=[
            pl.BlockSpec((BLK, 2 * D), lambda i: (i, 0)),
            pl.BlockSpec((1, 1), lambda i: (0, 0)),
        ],
        out_shape=[
            jax.ShapeDtypeStruct((USER_NUM, 2 * D), jnp.float32),
            jax.ShapeDtypeStruct((1, 1), jnp.float32),
        ],
    )(x, W1, b1, W2, b2, dW, db, labs)


def _spmm_body(qf_hbm, col_hbm, row1_hbm, val_hbm, zeros_hbm, out_hbm,
               qf_v, col_v, row128_v, val_v, y0_v, y1_v, acc0_sh, acc1_sh):
    c = lax.axis_index("c")
    s = lax.axis_index("s")
    wid = s * NC + c

    @pl.when(s == 0)
    def _():
        pltpu.sync_copy(zeros_hbm, acc0_sh)
        pltpu.sync_copy(zeros_hbm, acc1_sh)

    pltpu.sync_copy(qf_hbm, qf_v)
    plsc.subcore_barrier()

    base_e = wid * EPT

    def chunk_body(ci, _):
        e0 = base_e + ci * CHUNK
        pltpu.sync_copy(col_hbm.at[pl.ds(e0, CHUNK)], col_v)
        pltpu.sync_copy(val_hbm.at[pl.ds(e0, CHUNK)], val_v)

        def group_body(g, _):
            col2 = col_v[pl.ds(g * L, L)] * 2
            val = val_v[pl.ds(g * L, L)]
            g0 = plsc.load_gather(qf_v, [col2])
            g1 = plsc.load_gather(qf_v, [col2 + 1])
            y0_v[pl.ds(g * L, L)] = g0 * val
            y1_v[pl.ds(g * L, L)] = g1 * val
            return 0

        lax.fori_loop(0, CHUNK // L, group_body, 0, unroll=4)

        def sub_body(j, _):
            pltpu.sync_copy(row1_hbm.at[pl.ds(e0 + j * SUB, SUB)], row128_v)
            pltpu.sync_copy(y0_v.at[pl.ds(j * SUB, SUB)],
                            acc0_sh.at[row128_v], add=True)
            pltpu.sync_copy(y1_v.at[pl.ds(j * SUB, SUB)],
                            acc1_sh.at[row128_v], add=True)
            return 0

        lax.fori_loop(0, CHUNK_SUBS, sub_body, 0)
        return 0

    lax.fori_loop(0, NCHUNK, chunk_body, 0)
    plsc.subcore_barrier()

    @pl.when(s == 0)
    def _():
        pltpu.sync_copy(acc0_sh, out_hbm.at[c, 0])
        pltpu.sync_copy(acc1_sh, out_hbm.at[c, 1])


def _spmm_call(qf, col, row1, val, zeros):
    mesh = plsc.VectorSubcoreMesh(core_axis_name="c", subcore_axis_name="s", num_cores=NC, num_subcores=NS)
    f = pl.kernel(
        _spmm_body,
        out_type=jax.ShapeDtypeStruct((NC, 2, USER_NUM), jnp.float32),
        mesh=mesh,
        scratch_types=[
            pltpu.VMEM((2 * ITEM_NUM,), jnp.float32),
            pltpu.VMEM((CHUNK,), jnp.int32),
            pltpu.VMEM((SUB,), jnp.int32),
            pltpu.VMEM((CHUNK,), jnp.float32),
            pltpu.VMEM((CHUNK,), jnp.float32),
            pltpu.VMEM((CHUNK,), jnp.float32),
            pltpu.VMEM_SHARED((USER_NUM,), jnp.float32),
            pltpu.VMEM_SHARED((USER_NUM,), jnp.float32),
        ],
        compiler_params=pltpu.CompilerParams(needs_layout_passes=False, use_tc_tiling_on_sc=False),
    )
    return f(qf, col, row1, val, zeros)


def _gather_body(uf1_hbm, if1_hbm, uidx_hbm, iidx_hbm, ub_hbm, ib_hbm,
                 idx_v, rows_v, sem):
    c = lax.axis_index("c")
    s = lax.axis_index("s")
    wid = s * NC + c
    bpt = B // NW
    base = wid * bpt

    pltpu.sync_copy(uidx_hbm.at[pl.ds(base, bpt)], idx_v)
    pltpu.async_copy(uf1_hbm.at[idx_v], rows_v, sem).wait()
    pltpu.sync_copy(rows_v, ub_hbm.at[pl.ds(base, bpt)])

    pltpu.sync_copy(iidx_hbm.at[pl.ds(base, bpt)], idx_v)
    pltpu.async_copy(if1_hbm.at[idx_v], rows_v, sem).wait()
    pltpu.sync_copy(rows_v, ib_hbm.at[pl.ds(base, bpt)])


def _gather_call(uf1, if1, uidx, iidx):
    mesh = plsc.VectorSubcoreMesh(core_axis_name="c", subcore_axis_name="s", num_cores=NC, num_subcores=NS)
    f = pl.kernel(
        _gather_body,
        out_type=(
            jax.ShapeDtypeStruct((B, 2 * D), jnp.float32),
            jax.ShapeDtypeStruct((B, 2 * D), jnp.float32),
        ),
        mesh=mesh,
        scratch_types=[
            pltpu.VMEM((B // NW,), jnp.int32),
            pltpu.VMEM((B // NW, 2 * D), jnp.float32),
            pltpu.SemaphoreType.DMA,
        ],
        compiler_params=pltpu.CompilerParams(needs_layout_passes=False, use_tc_tiling_on_sc=False),
    )
    return f(uf1, if1, uidx, iidx)


def _final_body(acc_ref, lab_ref, db_ref, dsum_ref, ub_ref, ib_ref, rat_ref,
                out_ref, lsum_ref):
    i = pl.program_id(0)

    @pl.when(i == 0)
    def _():
        lsum_ref[0, 0] = 0.0

    ar = acc_ref[...]
    a0 = ar[0, 0, 0] + ar[2, 0, 0]
    a1 = ar[1, 0, 0] + ar[3, 0, 0]
    logits = jnp.stack([a0, a1], axis=1) + db_ref[...]
    lsum_ref[0, 0] += _loss_terms(logits, lab_ref[:, 0])

    @pl.when(i == NBLK - 1)
    def _():
        d_loss1 = dsum_ref[0, 0] * (1.0 / USER_NUM)
        d_loss1_l = lsum_ref[0, 0] * (1.0 / USER_NUM)
        ub = ub_ref[...]
        ib = ib_ref[...]
        pred = jnp.sum(ub * ib, axis=1, keepdims=True)
        loss_part = jnp.mean((pred - rat_ref[...]) ** 2)
        l2 = 0.01 * (jnp.sum(ub * ub) + jnp.sum(ib * ib)) * (1.0 / B)
        loss_p_square = loss_part + l2
        d_loss_all = d_loss1 + 0.5 * d_loss1_l
        out_ref[...] = jnp.stack(
            [d_loss_all, 10.0 * loss_p_square, -1000.0 * d_loss_all])[None, :]


def _final_call(acc, labs, db, dsum, ub, ib, rat):
    return pl.pallas_call(
        _final_body,
        grid=(NBLK,),
        in_specs=[
            pl.BlockSpec((2 * NC, 1, 1, BLK), lambda i: (0, i, 0, 0)),
            pl.BlockSpec((BLK, 3), lambda i: (i, 0)),
            pl.BlockSpec((1, 2), lambda i: (0, 0)),
            pl.BlockSpec((1, 1), lambda i: (0, 0)),
            pl.BlockSpec((B, 2 * D), lambda i: (0, 0)),
            pl.BlockSpec((B, 2 * D), lambda i: (0, 0)),
            pl.BlockSpec((B, 1), lambda i: (0, 0)),
        ],
        out_specs=pl.BlockSpec((1, 3), lambda i: (0, 0)),
        out_shape=jax.ShapeDtypeStruct((1, 3), jnp.float32),
        scratch_shapes=[pltpu.SMEM((1, 1), jnp.float32)],
    )(acc, labs, db, dsum, ub, ib, rat)


def kernel(adj_indices, adj_values, user_batch, rating_batch, item_batch,
           users_features, gcn_user_embs, gcn_item_embs,
           f1_W1, f1_b1, f1_W2, f1_b2,
           f2_W1, f2_b1, f2_W2, f2_b2,
           f3_W1, f3_b1, f3_W2, f3_b2,
           d1_W, d1_b, d2_W, d2_b, d3_W, d3_b):
    row = adj_indices[0].astype(jnp.int32)
    col = adj_indices[1].astype(jnp.int32)
    pad = E_PAD - E
    row_p = jnp.concatenate([row, jnp.zeros((pad,), jnp.int32)])
    col_p = jnp.concatenate([col, jnp.zeros((pad,), jnp.int32)])
    val_p = jnp.concatenate([adj_values, jnp.zeros((pad,), jnp.float32)])

    b1 = f1_b1[None, :]
    b2 = f1_b2[None, :]
    db = d1_b[None, :]
    labs = users_features.astype(jnp.int32)
    zeros_acc = jnp.zeros((USER_NUM,), jnp.float32)

    if1, q = _items_call(gcn_item_embs, f1_W1, b1, f1_W2, b2, d1_W)
    uf1, dsum = _users_call(gcn_user_embs, f1_W1, b1, f1_W2, b2, d1_W, db,
                            labs)
    acc = _spmm_call(q.reshape(-1), col_p, row_p, val_p, zeros_acc)
    ub, ib = _gather_call(uf1, if1, user_batch.astype(jnp.int32),
                          item_batch.astype(jnp.int32))
    out = _final_call(acc.reshape(2 * NC, NBLK, 1, BLK), labs, db, dsum, ub, ib,
                      rating_batch[:, None])
    return out[0]


# speedup vs baseline: 22.6684x; 1.0139x over previous
import jax
import jax.numpy as jnp
from jax import lax
from jax.experimental import pallas as pl
from jax.experimental.pallas import tpu as pltpu
from jax.experimental.pallas import tpu_sc as plsc

USER_NUM = 50000
ITEM_NUM = 50000
D = 64
H = 128
E = 800000
B = 4096

NC = 2
NS = 16
L = 16
NW = NC * NS

SUB = 128
EPT = 25600
E_PAD = EPT * NW
CHUNK_SUBS = 8
CHUNK = SUB * CHUNK_SUBS
NCHUNK = EPT // CHUNK

BLK = 2000
NBLK = USER_NUM // BLK

_HI = lax.Precision.HIGHEST


def _mlp(x, W1, b1, W2, b2):
    h = jnp.dot(x, W1, precision=_HI, preferred_element_type=jnp.float32) + b1
    h = jnp.where(h > 0, h, 0.2 * h)
    return jnp.dot(h, W2, precision=_HI, preferred_element_type=jnp.float32) + b2


def _loss_terms(logits, lab):
    l0 = logits[:, 0]
    l1 = logits[:, 1]
    m = jnp.maximum(l0, l1)
    logz = m + jnp.log(jnp.exp(l0 - m) + jnp.exp(l1 - m))
    ll = jnp.where(lab == 0, l0, l1)
    return jnp.sum(logz - ll)


def _items_body(x_ref, W1_ref, b1_ref, W2_ref, b2_ref, dW_ref,
                if1_ref, q_ref):
    f = _mlp(x_ref[...], W1_ref[...], b1_ref[...], W2_ref[...], b2_ref[...])
    if1_ref[...] = f
    q_ref[...] = jnp.dot(f, dW_ref[...], precision=_HI,
                         preferred_element_type=jnp.float32)


def _items_call(x, W1, b1, W2, b2, dW):
    return pl.pallas_call(
        _items_body,
        grid=(NBLK,),
        in_specs=[
            pl.BlockSpec((BLK, D), lambda i: (i, 0)),
            pl.BlockSpec((D, H), lambda i: (0, 0)),
            pl.BlockSpec((1, H), lambda i: (0, 0)),
            pl.BlockSpec((H, D), lambda i: (0, 0)),
            pl.BlockSpec((1, D), lambda i: (0, 0)),
            pl.BlockSpec((D, 2), lambda i: (0, 0)),
        ],
        out_specs=[
            pl.BlockSpec((BLK, D), lambda i: (i, 0)),
            pl.BlockSpec((BLK, 2), lambda i: (i, 0)),
        ],
        out_shape=[
            jax.ShapeDtypeStruct((ITEM_NUM, D), jnp.float32),
            jax.ShapeDtypeStruct((ITEM_NUM, 2), jnp.float32),
        ],
    )(x, W1, b1, W2, b2, dW)


def _users_body(x_ref, W1_ref, b1_ref, W2_ref, b2_ref, dW_ref, db_ref,
                lab_ref, uf1_ref, lsum_ref):
    i = pl.program_id(0)
    f = _mlp(x_ref[...], W1_ref[...], b1_ref[...], W2_ref[...], b2_ref[...])
    uf1_ref[...] = f
    logits = jnp.dot(f, dW_ref[...], precision=_HI,
                     preferred_element_type=jnp.float32) + db_ref[...]
    part = _loss_terms(logits, lab_ref[:, 0])

    @pl.when(i == 0)
    def _():
        lsum_ref[...] = jnp.zeros((1, 1), jnp.float32)

    lsum_ref[...] += jnp.reshape(part, (1, 1))


def _users_call(x, W1, b1, W2, b2, dW, db, labs):
    return pl.pallas_call(
        _users_body,
        grid=(NBLK,),
        in_specs=[
            pl.BlockSpec((BLK, D), lambda i: (i, 0)),
            pl.BlockSpec((D, H), lambda i: (0, 0)),
            pl.BlockSpec((1, H), lambda i: (0, 0)),
            pl.BlockSpec((H, D), lambda i: (0, 0)),
            pl.BlockSpec((1, D), lambda i: (0, 0)),
            pl.BlockSpec((D, 2), lambda i: (0, 0)),
            pl.BlockSpec((1, 2), lambda i: (0, 0)),
            pl.BlockSpec((BLK, 3), lambda i: (i, 0)),
        ],
        out_specs=[
            pl.BlockSpec((BLK, D), lambda i: (i, 0)),
            pl.BlockSpec((1, 1), lambda i: (0, 0)),
        ],
        out_shape=[
            jax.ShapeDtypeStruct((USER_NUM, D), jnp.float32),
            jax.ShapeDtypeStruct((1, 1), jnp.float32),
        ],
    )(x, W1, b1, W2, b2, dW, db, labs)


def _spmm_body(qf_hbm, col_hbm, row1_hbm, val_hbm, zeros_hbm, out_hbm,
               qf_v, col_v, val_v, y0_v, y1_v,
               r0_v, r1_v, r2_v, r3_v, r4_v, r5_v, r6_v, r7_v,
               acc0_sh, acc1_sh, sem_e, sem_r, sem_s):
    c = lax.axis_index("c")
    s = lax.axis_index("s")
    wid = s * NC + c
    rrefs = [r0_v, r1_v, r2_v, r3_v, r4_v, r5_v, r6_v, r7_v]

    @pl.when(s == 0)
    def _():
        pltpu.sync_copy(zeros_hbm, acc0_sh)
        pltpu.sync_copy(zeros_hbm, acc1_sh)

    pltpu.sync_copy(qf_hbm, qf_v)
    plsc.subcore_barrier()

    base_e = wid * EPT

    def chunk_body(ci, _):
        e0 = base_e + ci * CHUNK
        de = pltpu.async_copy(col_hbm.at[pl.ds(e0, CHUNK)], col_v, sem_e)
        dv = pltpu.async_copy(val_hbm.at[pl.ds(e0, CHUNK)], val_v, sem_e)
        drs = [
            pltpu.async_copy(row1_hbm.at[pl.ds(e0 + j * SUB, SUB)],
                             rrefs[j], sem_r)
            for j in range(CHUNK_SUBS)
        ]
        de.wait()
        dv.wait()

        def group_body(g, _):
            col2 = col_v[pl.ds(g * L, L)] * 2
            val = val_v[pl.ds(g * L, L)]
            g0 = plsc.load_gather(qf_v, [col2])
            g1 = plsc.load_gather(qf_v, [col2 + 1])
            y0_v[pl.ds(g * L, L)] = g0 * val
            y1_v[pl.ds(g * L, L)] = g1 * val
            return 0

        lax.fori_loop(0, CHUNK // L, group_body, 0, unroll=4)

        for d in drs:
            d.wait()
        dss = []
        for j in range(CHUNK_SUBS):
            dss.append(pltpu.async_copy(
                y0_v.at[pl.ds(j * SUB, SUB)], acc0_sh.at[rrefs[j]], sem_s,
                add=True))
            dss.append(pltpu.async_copy(
                y1_v.at[pl.ds(j * SUB, SUB)], acc1_sh.at[rrefs[j]], sem_s,
                add=True))
        for d in dss:
            d.wait()
        return 0

    lax.fori_loop(0, NCHUNK, chunk_body, 0)
    plsc.subcore_barrier()

    @pl.when(s == 0)
    def _():
        pltpu.sync_copy(acc0_sh, out_hbm.at[c, 0])
        pltpu.sync_copy(acc1_sh, out_hbm.at[c, 1])


def _spmm_call(qf, col, row1, val, zeros):
    mesh = plsc.VectorSubcoreMesh(core_axis_name="c", subcore_axis_name="s", num_cores=NC, num_subcores=NS)
    f = pl.kernel(
        _spmm_body,
        out_type=jax.ShapeDtypeStruct((NC, 2, USER_NUM), jnp.float32),
        mesh=mesh,
        scratch_types=(
            [
                pltpu.VMEM((2 * ITEM_NUM,), jnp.float32),
                pltpu.VMEM((CHUNK,), jnp.int32),
                pltpu.VMEM((CHUNK,), jnp.float32),
                pltpu.VMEM((CHUNK,), jnp.float32),
                pltpu.VMEM((CHUNK,), jnp.float32),
            ]
            + [pltpu.VMEM((SUB,), jnp.int32) for _ in range(CHUNK_SUBS)]
            + [
                pltpu.VMEM_SHARED((USER_NUM,), jnp.float32),
                pltpu.VMEM_SHARED((USER_NUM,), jnp.float32),
                pltpu.SemaphoreType.DMA,
                pltpu.SemaphoreType.DMA,
                pltpu.SemaphoreType.DMA,
            ]
        ),
        compiler_params=pltpu.CompilerParams(needs_layout_passes=False, use_tc_tiling_on_sc=False),
    )
    return f(qf, col, row1, val, zeros)


def _gather_body(uf1_hbm, if1_hbm, uidx_hbm, iidx_hbm, ub_hbm, ib_hbm,
                 idx_v, rows_v, sem):
    c = lax.axis_index("c")
    s = lax.axis_index("s")
    wid = s * NC + c
    bpt = B // NW
    base = wid * bpt

    pltpu.sync_copy(uidx_hbm.at[pl.ds(base, bpt)], idx_v)
    pltpu.async_copy(uf1_hbm.at[idx_v], rows_v, sem).wait()
    pltpu.sync_copy(rows_v, ub_hbm.at[pl.ds(base, bpt)])

    pltpu.sync_copy(iidx_hbm.at[pl.ds(base, bpt)], idx_v)
    pltpu.async_copy(if1_hbm.at[idx_v], rows_v, sem).wait()
    pltpu.sync_copy(rows_v, ib_hbm.at[pl.ds(base, bpt)])


def _gather_call(uf1, if1, uidx, iidx):
    mesh = plsc.VectorSubcoreMesh(core_axis_name="c", subcore_axis_name="s", num_cores=NC, num_subcores=NS)
    f = pl.kernel(
        _gather_body,
        out_type=(
            jax.ShapeDtypeStruct((B, D), jnp.float32),
            jax.ShapeDtypeStruct((B, D), jnp.float32),
        ),
        mesh=mesh,
        scratch_types=[
            pltpu.VMEM((B // NW,), jnp.int32),
            pltpu.VMEM((B // NW, D), jnp.float32),
            pltpu.SemaphoreType.DMA,
        ],
        compiler_params=pltpu.CompilerParams(needs_layout_passes=False, use_tc_tiling_on_sc=False),
    )
    return f(uf1, if1, uidx, iidx)


def _final_body(acc_ref, lab_ref, db_ref, dsum_ref, ub_ref, ib_ref, rat_ref,
                out_ref, lsum_ref):
    i = pl.program_id(0)

    @pl.when(i == 0)
    def _():
        lsum_ref[0, 0] = 0.0

    ar = acc_ref[...]
    a0 = ar[0, 0, 0] + ar[2, 0, 0]
    a1 = ar[1, 0, 0] + ar[3, 0, 0]
    logits = jnp.stack([a0, a1], axis=1) + db_ref[...]
    lsum_ref[0, 0] += _loss_terms(logits, lab_ref[:, 0])

    @pl.when(i == NBLK - 1)
    def _():
        d_loss1 = dsum_ref[0, 0] * (1.0 / USER_NUM)
        d_loss1_l = lsum_ref[0, 0] * (1.0 / USER_NUM)
        ub = ub_ref[...]
        ib = ib_ref[...]
        pred = jnp.sum(ub * ib, axis=1, keepdims=True)
        loss_part = jnp.mean((pred - rat_ref[...]) ** 2)
        l2 = 0.01 * (jnp.sum(ub * ub) + jnp.sum(ib * ib)) * (1.0 / B)
        loss_p_square = loss_part + l2
        d_loss_all = d_loss1 + 0.5 * d_loss1_l
        out_ref[...] = jnp.stack(
            [d_loss_all, 10.0 * loss_p_square, -1000.0 * d_loss_all])[None, :]


def _final_call(acc, labs, db, dsum, ub, ib, rat):
    return pl.pallas_call(
        _final_body,
        grid=(NBLK,),
        in_specs=[
            pl.BlockSpec((2 * NC, 1, 1, BLK), lambda i: (0, i, 0, 0)),
            pl.BlockSpec((BLK, 3), lambda i: (i, 0)),
            pl.BlockSpec((1, 2), lambda i: (0, 0)),
            pl.BlockSpec((1, 1), lambda i: (0, 0)),
            pl.BlockSpec((B, D), lambda i: (0, 0)),
            pl.BlockSpec((B, D), lambda i: (0, 0)),
            pl.BlockSpec((B, 1), lambda i: (0, 0)),
        ],
        out_specs=pl.BlockSpec((1, 3), lambda i: (0, 0)),
        out_shape=jax.ShapeDtypeStruct((1, 3), jnp.float32),
        scratch_shapes=[pltpu.SMEM((1, 1), jnp.float32)],
    )(acc, labs, db, dsum, ub, ib, rat)


def kernel(adj_indices, adj_values, user_batch, rating_batch, item_batch,
           users_features, gcn_user_embs, gcn_item_embs,
           f1_W1, f1_b1, f1_W2, f1_b2,
           f2_W1, f2_b1, f2_W2, f2_b2,
           f3_W1, f3_b1, f3_W2, f3_b2,
           d1_W, d1_b, d2_W, d2_b, d3_W, d3_b):
    row = adj_indices[0].astype(jnp.int32)
    col = adj_indices[1].astype(jnp.int32)
    pad = E_PAD - E
    row_p = jnp.concatenate([row, jnp.zeros((pad,), jnp.int32)])
    col_p = jnp.concatenate([col, jnp.zeros((pad,), jnp.int32)])
    val_p = jnp.concatenate([adj_values, jnp.zeros((pad,), jnp.float32)])

    b1 = f1_b1[None, :]
    b2 = f1_b2[None, :]
    db = d1_b[None, :]
    labs = users_features.astype(jnp.int32)
    zeros_acc = jnp.zeros((USER_NUM,), jnp.float32)

    if1, q = _items_call(gcn_item_embs, f1_W1, b1, f1_W2, b2, d1_W)
    uf1, dsum = _users_call(gcn_user_embs, f1_W1, b1, f1_W2, b2, d1_W, db,
                            labs)
    acc = _spmm_call(q.reshape(-1), col_p, row_p, val_p, zeros_acc)
    ub, ib = _gather_call(uf1, if1, user_batch.astype(jnp.int32),
                          item_batch.astype(jnp.int32))
    out = _final_call(acc.reshape(2 * NC, NBLK, 1, BLK), labs, db, dsum, ub, ib,
                      rating_batch[:, None])
    return out[0]
